# baseline (device time: 71698 ns/iter reference)
import jax
import jax.numpy as jnp
from jax import lax
from jax.experimental import pallas as pl
from jax.experimental.pallas import tpu as pltpu

N_DEV = 32
LAG = 4


def kernel(x, w_mat):
    m_per, k = x.shape
    _, n_total = w_mat.shape
    n_chunk = n_total // N_DEV

    def body(x_ref, w_hbm, out_ref, w_buf, stage, inbox,
             w_sems, send_sems, recv_sems):
        my = lax.axis_index("i")

        def w_copy(t):
            j = (my + t) % N_DEV
            return pltpu.make_async_copy(
                w_hbm.at[:, pl.ds(j * n_chunk, n_chunk)],
                w_buf.at[t % 2],
                w_sems.at[t % 2],
            )

        def harvest(t):
            s = (my - t) % N_DEV
            recv = pltpu.make_async_remote_copy(
                src_ref=stage.at[t],
                dst_ref=inbox.at[s],
                send_sem=send_sems.at[t],
                recv_sem=recv_sems.at[s],
                device_id=(s,),
                device_id_type=pl.DeviceIdType.MESH,
            )
            recv.wait_recv()
            out_ref[pl.ds(s * m_per, m_per), :] = inbox[s].astype(jnp.float32)

        w_copy(0).start()
        xv = x_ref[...].astype(jnp.bfloat16)

        sends = []
        for t in range(N_DEV):
            if t + 1 < N_DEV:
                w_copy(t + 1).start()
            w_copy(t).wait()
            chunk = jnp.dot(xv, w_buf[t % 2].astype(jnp.bfloat16),
                            preferred_element_type=jnp.float32)
            if t == 0:
                out_ref[pl.ds(my * m_per, m_per), :] = chunk
            else:
                stage[t] = chunk.astype(jnp.bfloat16)
                rdma = pltpu.make_async_remote_copy(
                    src_ref=stage.at[t],
                    dst_ref=inbox.at[my],
                    send_sem=send_sems.at[t],
                    recv_sem=recv_sems.at[my],
                    device_id=((my + t) % N_DEV,),
                    device_id_type=pl.DeviceIdType.MESH,
                )
                rdma.start()
                sends.append(rdma)
            if t >= LAG:
                harvest(t - LAG + 1)

        for t in range(N_DEV - LAG + 1, N_DEV):
            harvest(t)

        for rdma in sends:
            rdma.wait_send()

    return pl.pallas_call(
        body,
        out_shape=jax.ShapeDtypeStruct((N_DEV * m_per, n_chunk), jnp.float32),
        in_specs=[
            pl.BlockSpec(memory_space=pltpu.VMEM),
            pl.BlockSpec(memory_space=pltpu.MemorySpace.HBM),
        ],
        out_specs=pl.BlockSpec(memory_space=pltpu.VMEM),
        scratch_shapes=[
            pltpu.VMEM((2, k, n_chunk), jnp.float32),
            pltpu.VMEM((N_DEV, m_per, n_chunk), jnp.bfloat16),
            pltpu.VMEM((N_DEV, m_per, n_chunk), jnp.bfloat16),
            pltpu.SemaphoreType.DMA((2,)),
            pltpu.SemaphoreType.DMA((N_DEV,)),
            pltpu.SemaphoreType.DMA((N_DEV,)),
        ],
    )(x, w_mat)


# device time: 63593 ns/iter; 1.1275x vs baseline; 1.1275x over previous
import jax
import jax.numpy as jnp
from jax import lax
from jax.experimental import pallas as pl
from jax.experimental.pallas import tpu as pltpu

N_DEV = 32
CPP = 2
N_PHASE = N_DEV // CPP
LAG_CHUNKS = 8


def kernel(x, w_mat):
    m_per, k = x.shape
    _, n_total = w_mat.shape
    n_chunk = n_total // N_DEV

    def body(x_ref, w_hbm, out_ref, w_buf, stage, inbox,
             w_sems, send_sems, recv_sems):
        my = lax.axis_index("i")

        def w_copy(d, slot, q):
            j = (my + d) % N_DEV
            return pltpu.make_async_copy(
                w_hbm.at[:, pl.ds(j * n_chunk, n_chunk)],
                w_buf.at[slot, :, pl.ds(q * n_chunk, n_chunk)],
                w_sems.at[slot, q],
            )

        def start_phase_fetch(p):
            for q in range(CPP):
                w_copy(CPP * p + q, p % 2, q).start()

        def wait_phase_fetch(p):
            for q in range(CPP):
                w_copy(CPP * p + q, p % 2, q).wait()

        def harvest(d):
            s = (my - d) % N_DEV
            recv = pltpu.make_async_remote_copy(
                src_ref=stage.at[d],
                dst_ref=inbox.at[s],
                send_sem=send_sems.at[d],
                recv_sem=recv_sems.at[s],
                device_id=(s,),
                device_id_type=pl.DeviceIdType.MESH,
            )
            recv.wait_recv()
            out_ref[pl.ds(s * m_per, m_per), :] = inbox[s].astype(jnp.float32)

        start_phase_fetch(0)
        xv = x_ref[...].astype(jnp.bfloat16)

        sends = []
        harvested = 0
        for p in range(N_PHASE):
            if p + 1 < N_PHASE:
                start_phase_fetch(p + 1)
            wait_phase_fetch(p)
            block = jnp.dot(xv, w_buf[p % 2].astype(jnp.bfloat16),
                            preferred_element_type=jnp.float32)
            for q in range(CPP):
                d = CPP * p + q
                piece = block[:, q * n_chunk:(q + 1) * n_chunk]
                if d == 0:
                    out_ref[pl.ds(my * m_per, m_per), :] = piece
                else:
                    stage[d] = piece.astype(jnp.bfloat16)
                    rdma = pltpu.make_async_remote_copy(
                        src_ref=stage.at[d],
                        dst_ref=inbox.at[my],
                        send_sem=send_sems.at[d],
                        recv_sem=recv_sems.at[my],
                        device_id=((my + d) % N_DEV,),
                        device_id_type=pl.DeviceIdType.MESH,
                    )
                    rdma.start()
                    sends.append(rdma)
            front = CPP * (p + 1) - 1
            while harvested + 1 <= front - LAG_CHUNKS:
                harvested += 1
                harvest(harvested)

        while harvested + 1 < N_DEV:
            harvested += 1
            harvest(harvested)

        for rdma in sends:
            rdma.wait_send()

    return pl.pallas_call(
        body,
        out_shape=jax.ShapeDtypeStruct((N_DEV * m_per, n_chunk), jnp.float32),
        in_specs=[
            pl.BlockSpec(memory_space=pltpu.VMEM),
            pl.BlockSpec(memory_space=pltpu.MemorySpace.HBM),
        ],
        out_specs=pl.BlockSpec(memory_space=pltpu.VMEM),
        scratch_shapes=[
            pltpu.VMEM((2, k, CPP * n_chunk), jnp.float32),
            pltpu.VMEM((N_DEV, m_per, n_chunk), jnp.bfloat16),
            pltpu.VMEM((N_DEV, m_per, n_chunk), jnp.bfloat16),
            pltpu.SemaphoreType.DMA((2, CPP)),
            pltpu.SemaphoreType.DMA((N_DEV,)),
            pltpu.SemaphoreType.DMA((N_DEV,)),
        ],
    )(x, w_mat)
